# baseline (speedup 1.0000x reference)
import functools

import jax
import jax.numpy as jnp
from jax import lax
from jax.experimental import pallas as pl
from jax.experimental.pallas import tpu as pltpu
from jax.experimental.pallas import tpu_sc as plsc

V = 32
C = 128
L = 16
KC = 8
M_SC = 8
BS = 4


def _make_sc_body(rounds):
    def _sc_body(counts_hbm, pc_hbm, out_hbm, rs_hbm, buf_a, buf_b, pc_v,
                 rs_buf, in_a, in_b, out_a, out_b):
        c = lax.axis_index("c")
        s = lax.axis_index("s")
        k0 = s * KC

        pltpu.sync_copy(pc_hbm, pc_v)
        pc_s = pc_v[...][0]
        lane_ids = lax.iota(jnp.int32, L)

        def lane_total(vec):
            vals = [vec[i] for i in range(L)]
            while len(vals) > 1:
                vals = [a + b for a, b in zip(vals[::2], vals[1::2])]
            return vals[0]

        def in_slice(r):
            i = r * 2 + c
            return counts_hbm.at[i, :, pl.ds(k0, KC), :]

        def out_slice(r):
            i = r * 2 + c
            return out_hbm.at[i, :, pl.ds(k0, KC), :]

        def compute(buf, r):
            rs_vec = jnp.zeros((L,), jnp.float32)
            for k in range(KC):
                def sum_j(j, acc):
                    t = acc
                    for lv in range(C // L):
                        t = t + buf[j, k, pl.ds(lv * L, L)]
                    return t

                acc = lax.fori_loop(0, V, sum_j, jnp.zeros((L,), jnp.float32))
                rs_val = lane_total(acc) + pc_s * jnp.float32(V * C)
                rs_val_vec = jnp.full((L,), rs_val)
                denom = jnp.where(rs_val_vec > 0, rs_val_vec, jnp.float32(1.0))
                recip = jnp.full((L,), 1.0, jnp.float32) / denom
                pr = pc_v[...] * recip
                rs_vec = jnp.where(lane_ids == k, rs_val_vec, rs_vec)

                def scale_j(j, carry):
                    for lv in range(C // L):
                        sl = pl.ds(lv * L, L)
                        buf[j, k, sl] = buf[j, k, sl] * recip + pr
                    return carry

                lax.fori_loop(0, V, scale_j, 0)
            rs_buf[r] = rs_vec

        pltpu.async_copy(in_slice(0), buf_a, in_a)
        pltpu.async_copy(in_slice(1), buf_b, in_b)

        def round_pair(rr, carry):
            r0 = rr * 2
            r1 = r0 + 1
            pltpu.make_async_copy(in_slice(r0), buf_a, in_a).wait()
            compute(buf_a, r0)
            oa = pltpu.async_copy(buf_a, out_slice(r0), out_a)
            pltpu.make_async_copy(in_slice(r1), buf_b, in_b).wait()
            compute(buf_b, r1)
            ob = pltpu.async_copy(buf_b, out_slice(r1), out_b)

            @pl.when(rr < rounds // 2 - 1)
            def _prefetch():
                oa.wait()
                pltpu.async_copy(in_slice(r0 + 2), buf_a, in_a)
                ob.wait()
                pltpu.async_copy(in_slice(r1 + 2), buf_b, in_b)

            return carry

        lax.fori_loop(0, rounds // 2, round_pair, 0)
        pltpu.make_async_copy(buf_a, out_slice(rounds - 2), out_a).wait()
        pltpu.make_async_copy(buf_b, out_slice(rounds - 1), out_b).wait()
        pltpu.sync_copy(rs_buf, rs_hbm.at[:, c, s, :])

    return _sc_body


def _sc_normalize(counts_sc, pcv):
    rounds = M_SC // 2
    mesh = plsc.VectorSubcoreMesh(core_axis_name="c", subcore_axis_name="s")
    k = functools.partial(
        pl.kernel,
        mesh=mesh,
        out_type=[
            jax.ShapeDtypeStruct((M_SC, V, C, C), jnp.float32),
            jax.ShapeDtypeStruct((rounds, 2, V // 2, L), jnp.float32),
        ],
        scratch_types=[
            pltpu.VMEM((V, KC, C), jnp.float32),
            pltpu.VMEM((V, KC, C), jnp.float32),
            pltpu.VMEM((L,), jnp.float32),
            pltpu.VMEM((rounds, L), jnp.float32),
            pltpu.SemaphoreType.DMA,
            pltpu.SemaphoreType.DMA,
            pltpu.SemaphoreType.DMA,
            pltpu.SemaphoreType.DMA,
        ],
    )(_make_sc_body(rounds))
    out, rs = k(counts_sc, pcv)
    return out.reshape(-1), rs[..., :KC].reshape(-1)


def _tc_block(pc_ref, counts_ref, out_ref, rs_ref):
    x = counts_ref[...]
    pc = pc_ref[0, 0]
    rs = x.sum(axis=1).sum(axis=2) + pc * jnp.float32(V * C)
    denom = jnp.where(rs > 0, rs, jnp.float32(1.0))
    recip = jnp.float32(1.0) / denom
    out_ref[...] = x * recip[:, None, :, None] + (pc * recip)[:, None, :, None]
    rs_ref[:, 0, :] = rs


def _tc_normalize(counts_tc, pc2d):
    n = counts_tc.shape[0]
    out, rs = pl.pallas_call(
        _tc_block,
        grid=(n // BS,),
        in_specs=[
            pl.BlockSpec(memory_space=pltpu.SMEM),
            pl.BlockSpec((BS, V, C, C), lambda i: (i, 0, 0, 0)),
        ],
        out_specs=[
            pl.BlockSpec((BS, V, C, C), lambda i: (i, 0, 0, 0)),
            pl.BlockSpec((BS, 1, C), lambda i: (i, 0, 0)),
        ],
        out_shape=[
            jax.ShapeDtypeStruct((n, V, C, C), jnp.float32),
            jax.ShapeDtypeStruct((n, 1, C), jnp.float32),
        ],
    )(pc2d, counts_tc)
    return out.reshape(-1), rs.reshape(-1)


def kernel(transition_counts, pseudocount, hidden_states):
    del hidden_states
    counts = transition_counts.reshape(V, V, C, C)
    pcv = jnp.full((L,), pseudocount, jnp.float32)
    pc2d = jnp.asarray(pseudocount, jnp.float32).reshape(1, 1)
    sc_out, sc_rs = _sc_normalize(counts[:M_SC], pcv)
    tc_out, tc_rs = _tc_normalize(counts[M_SC:], pc2d)
    out = jnp.concatenate([sc_out, tc_out])
    rs = jnp.concatenate([sc_rs, tc_rs])
    return out, rs


# speedup vs baseline: 1.5407x; 1.5407x over previous
import functools

import jax
import jax.numpy as jnp
from jax import lax
from jax.experimental import pallas as pl
from jax.experimental.pallas import tpu as pltpu
from jax.experimental.pallas import tpu_sc as plsc

V = 32
C = 128
L = 16
KC = 8
M_SC = 8
BS = 4


def _make_sc_body(rounds):
    def _sc_body(counts_hbm, pc_hbm, out_hbm, rs_hbm, buf_a, buf_b, pc_v,
                 rs_buf, in_a, in_b, out_a, out_b):
        c = lax.axis_index("c")
        s = lax.axis_index("s")
        k0 = s * KC

        pltpu.sync_copy(pc_hbm, pc_v)
        pc_s = pc_v[...][0]
        lane_ids = lax.iota(jnp.int32, L)

        def lane_total(vec):
            vals = [vec[i] for i in range(L)]
            while len(vals) > 1:
                vals = [a + b for a, b in zip(vals[::2], vals[1::2])]
            return vals[0]

        def in_slice(r):
            i = r * 2 + c
            return counts_hbm.at[i, :, pl.ds(k0, KC), :]

        def out_slice(r):
            i = r * 2 + c
            return out_hbm.at[i, :, pl.ds(k0, KC), :]

        def compute(buf, r):
            rs_vec = jnp.zeros((L,), jnp.float32)
            for k in range(KC):
                def sum_j(j, acc):
                    t = acc
                    for lv in range(C // L):
                        t = t + buf[j, k, pl.ds(lv * L, L)]
                    return t

                acc = lax.fori_loop(0, V, sum_j, jnp.zeros((L,), jnp.float32))
                rs_val = lane_total(acc) + pc_s * jnp.float32(V * C)
                rs_val_vec = jnp.full((L,), rs_val)
                denom = jnp.where(rs_val_vec > 0, rs_val_vec, jnp.float32(1.0))
                recip = jnp.full((L,), 1.0, jnp.float32) / denom
                pr = pc_v[...] * recip
                rs_vec = jnp.where(lane_ids == k, rs_val_vec, rs_vec)

                def scale_j(j, carry):
                    for lv in range(C // L):
                        sl = pl.ds(lv * L, L)
                        buf[j, k, sl] = buf[j, k, sl] * recip + pr
                    return carry

                lax.fori_loop(0, V, scale_j, 0)
            rs_buf[r] = rs_vec

        pltpu.async_copy(in_slice(0), buf_a, in_a)
        pltpu.async_copy(in_slice(1), buf_b, in_b)

        def round_pair(rr, carry):
            r0 = rr * 2
            r1 = r0 + 1
            pltpu.make_async_copy(in_slice(r0), buf_a, in_a).wait()
            compute(buf_a, r0)
            oa = pltpu.async_copy(buf_a, out_slice(r0), out_a)
            pltpu.make_async_copy(in_slice(r1), buf_b, in_b).wait()
            compute(buf_b, r1)
            ob = pltpu.async_copy(buf_b, out_slice(r1), out_b)

            @pl.when(rr < rounds // 2 - 1)
            def _prefetch():
                oa.wait()
                pltpu.async_copy(in_slice(r0 + 2), buf_a, in_a)
                ob.wait()
                pltpu.async_copy(in_slice(r1 + 2), buf_b, in_b)

            return carry

        lax.fori_loop(0, rounds // 2, round_pair, 0)
        pltpu.make_async_copy(buf_a, out_slice(rounds - 2), out_a).wait()
        pltpu.make_async_copy(buf_b, out_slice(rounds - 1), out_b).wait()
        pltpu.sync_copy(rs_buf, rs_hbm.at[:, c, s, :])

    return _sc_body


def _sc_normalize(counts_sc, pcv):
    rounds = M_SC // 2
    mesh = plsc.VectorSubcoreMesh(core_axis_name="c", subcore_axis_name="s")
    k = functools.partial(
        pl.kernel,
        mesh=mesh,
        out_type=[
            jax.ShapeDtypeStruct((V, V, C, C), jnp.float32),
            jax.ShapeDtypeStruct((rounds, 2, V // 2, L), jnp.float32),
        ],
        scratch_types=[
            pltpu.VMEM((V, KC, C), jnp.float32),
            pltpu.VMEM((V, KC, C), jnp.float32),
            pltpu.VMEM((L,), jnp.float32),
            pltpu.VMEM((rounds, L), jnp.float32),
            pltpu.SemaphoreType.DMA,
            pltpu.SemaphoreType.DMA,
            pltpu.SemaphoreType.DMA,
            pltpu.SemaphoreType.DMA,
        ],
    )(_make_sc_body(rounds))
    out, rs = k(counts_sc, pcv)
    return out, rs[..., :KC].reshape(-1)


def _tc_block(pc_ref, counts_ref, out_ref, rs_ref):
    x = counts_ref[...]
    pc = pc_ref[0, 0]
    rs = x.sum(axis=1).sum(axis=2) + pc * jnp.float32(V * C)
    denom = jnp.where(rs > 0, rs, jnp.float32(1.0))
    recip = jnp.float32(1.0) / denom
    out_ref[...] = x * recip[:, None, :, None] + (pc * recip)[:, None, :, None]
    rs_ref[:, 0, :] = rs


def _tc_block_alias(pc_ref, counts_ref, sc_ref, out_ref, rs_ref):
    del sc_ref
    _tc_block(pc_ref, counts_ref, out_ref, rs_ref)


def _tc_normalize(counts, pc2d, sc_out):
    n = V - M_SC
    off = M_SC // BS
    out, rs = pl.pallas_call(
        _tc_block_alias,
        grid=(n // BS,),
        in_specs=[
            pl.BlockSpec(memory_space=pltpu.SMEM),
            pl.BlockSpec((BS, V, C, C), lambda i: (i + off, 0, 0, 0)),
            pl.BlockSpec(memory_space=pl.ANY),
        ],
        out_specs=[
            pl.BlockSpec((BS, V, C, C), lambda i: (i + off, 0, 0, 0)),
            pl.BlockSpec((BS, 1, C), lambda i: (i, 0, 0)),
        ],
        out_shape=[
            jax.ShapeDtypeStruct((V, V, C, C), jnp.float32),
            jax.ShapeDtypeStruct((n, 1, C), jnp.float32),
        ],
        input_output_aliases={2: 0},
    )(pc2d, counts, sc_out)
    return out.reshape(-1), rs.reshape(-1)


def kernel(transition_counts, pseudocount, hidden_states):
    del hidden_states
    counts = transition_counts.reshape(V, V, C, C)
    pcv = jnp.full((L,), pseudocount, jnp.float32)
    pc2d = jnp.asarray(pseudocount, jnp.float32).reshape(1, 1)
    sc_out, sc_rs = _sc_normalize(counts[:M_SC], pcv)
    out, tc_rs = _tc_normalize(counts, pc2d, sc_out)
    rs = jnp.concatenate([sc_rs, tc_rs])
    return out, rs


# speedup vs baseline: 1.7999x; 1.1682x over previous
import functools

import jax
import jax.numpy as jnp
from jax import lax
from jax.experimental import pallas as pl
from jax.experimental.pallas import tpu as pltpu
from jax.experimental.pallas import tpu_sc as plsc

V = 32
C = 128
L = 16
KC = 8
M_SC = 4
BS = 4


def _make_sc_body(rounds):
    def _sc_body(counts_hbm, pc_hbm, out_hbm, rs_hbm, buf_a, buf_b, pc_v,
                 rs_buf, in_a, in_b, out_a, out_b):
        c = lax.axis_index("c")
        s = lax.axis_index("s")
        k0 = s * KC

        pltpu.sync_copy(pc_hbm, pc_v)
        pc_s = pc_v[...][0]
        lane_ids = lax.iota(jnp.int32, L)

        def lane_total(vec):
            vals = [vec[i] for i in range(L)]
            while len(vals) > 1:
                vals = [a + b for a, b in zip(vals[::2], vals[1::2])]
            return vals[0]

        def in_slice(r):
            i = r * 2 + c
            return counts_hbm.at[i, :, pl.ds(k0, KC), :]

        def out_slice(r):
            i = r * 2 + c
            return out_hbm.at[i, :, pl.ds(k0, KC), :]

        def compute(buf, r):
            rs_vec = jnp.zeros((L,), jnp.float32)
            for k in range(KC):
                def sum_j(j, acc):
                    t = acc
                    for lv in range(C // L):
                        t = t + buf[j, k, pl.ds(lv * L, L)]
                    return t

                acc = lax.fori_loop(0, V, sum_j, jnp.zeros((L,), jnp.float32))
                rs_val = lane_total(acc) + pc_s * jnp.float32(V * C)
                rs_val_vec = jnp.full((L,), rs_val)
                denom = jnp.where(rs_val_vec > 0, rs_val_vec, jnp.float32(1.0))
                recip = jnp.full((L,), 1.0, jnp.float32) / denom
                pr = pc_v[...] * recip
                rs_vec = jnp.where(lane_ids == k, rs_val_vec, rs_vec)

                def scale_j(j, carry):
                    for lv in range(C // L):
                        sl = pl.ds(lv * L, L)
                        buf[j, k, sl] = buf[j, k, sl] * recip + pr
                    return carry

                lax.fori_loop(0, V, scale_j, 0)
            rs_buf[r] = rs_vec

        pltpu.async_copy(in_slice(0), buf_a, in_a)
        pltpu.async_copy(in_slice(1), buf_b, in_b)

        def round_pair(rr, carry):
            r0 = rr * 2
            r1 = r0 + 1
            pltpu.make_async_copy(in_slice(r0), buf_a, in_a).wait()
            compute(buf_a, r0)
            oa = pltpu.async_copy(buf_a, out_slice(r0), out_a)
            pltpu.make_async_copy(in_slice(r1), buf_b, in_b).wait()
            compute(buf_b, r1)
            ob = pltpu.async_copy(buf_b, out_slice(r1), out_b)

            @pl.when(rr < rounds // 2 - 1)
            def _prefetch():
                oa.wait()
                pltpu.async_copy(in_slice(r0 + 2), buf_a, in_a)
                ob.wait()
                pltpu.async_copy(in_slice(r1 + 2), buf_b, in_b)

            return carry

        lax.fori_loop(0, rounds // 2, round_pair, 0)
        pltpu.make_async_copy(buf_a, out_slice(rounds - 2), out_a).wait()
        pltpu.make_async_copy(buf_b, out_slice(rounds - 1), out_b).wait()
        pltpu.sync_copy(rs_buf, rs_hbm.at[:, c, s, :])

    return _sc_body


def _sc_normalize(counts_sc, pcv):
    rounds = M_SC // 2
    mesh = plsc.VectorSubcoreMesh(core_axis_name="c", subcore_axis_name="s")
    k = functools.partial(
        pl.kernel,
        mesh=mesh,
        out_type=[
            jax.ShapeDtypeStruct((V, V, C, C), jnp.float32),
            jax.ShapeDtypeStruct((rounds, 2, V // 2, L), jnp.float32),
        ],
        scratch_types=[
            pltpu.VMEM((V, KC, C), jnp.float32),
            pltpu.VMEM((V, KC, C), jnp.float32),
            pltpu.VMEM((L,), jnp.float32),
            pltpu.VMEM((rounds, L), jnp.float32),
            pltpu.SemaphoreType.DMA,
            pltpu.SemaphoreType.DMA,
            pltpu.SemaphoreType.DMA,
            pltpu.SemaphoreType.DMA,
        ],
    )(_make_sc_body(rounds))
    out, rs = k(counts_sc, pcv)
    return out, rs[..., :KC].reshape(-1)


def _tc_block(pc_ref, counts_ref, out_ref, rs_ref):
    x = counts_ref[...]
    pc = pc_ref[0, 0]
    rs = x.sum(axis=1).sum(axis=2) + pc * jnp.float32(V * C)
    denom = jnp.where(rs > 0, rs, jnp.float32(1.0))
    recip = jnp.float32(1.0) / denom
    out_ref[...] = x * recip[:, None, :, None] + (pc * recip)[:, None, :, None]
    rs_ref[:, 0, :] = rs


def _tc_block_alias(pc_ref, counts_ref, sc_ref, out_ref, rs_ref):
    del sc_ref
    _tc_block(pc_ref, counts_ref, out_ref, rs_ref)


def _tc_normalize(counts, pc2d, sc_out):
    n = V - M_SC
    off = M_SC // BS
    out, rs = pl.pallas_call(
        _tc_block_alias,
        grid=(n // BS,),
        in_specs=[
            pl.BlockSpec(memory_space=pltpu.SMEM),
            pl.BlockSpec((BS, V, C, C), lambda i: (i + off, 0, 0, 0)),
            pl.BlockSpec(memory_space=pl.ANY),
        ],
        out_specs=[
            pl.BlockSpec((BS, V, C, C), lambda i: (i + off, 0, 0, 0)),
            pl.BlockSpec((BS, 1, C), lambda i: (i, 0, 0)),
        ],
        out_shape=[
            jax.ShapeDtypeStruct((V, V, C, C), jnp.float32),
            jax.ShapeDtypeStruct((n, 1, C), jnp.float32),
        ],
        input_output_aliases={2: 0},
    )(pc2d, counts, sc_out)
    return out.reshape(-1), rs.reshape(-1)


def kernel(transition_counts, pseudocount, hidden_states):
    del hidden_states
    counts = transition_counts.reshape(V, V, C, C)
    pcv = jnp.full((L,), pseudocount, jnp.float32)
    pc2d = jnp.asarray(pseudocount, jnp.float32).reshape(1, 1)
    sc_out, sc_rs = _sc_normalize(counts[:M_SC], pcv)
    out, tc_rs = _tc_normalize(counts, pc2d, sc_out)
    rs = jnp.concatenate([sc_rs, tc_rs])
    return out, rs


# speedup vs baseline: 1.9461x; 1.0812x over previous
import functools

import jax
import jax.numpy as jnp
from jax import lax
from jax.experimental import pallas as pl
from jax.experimental.pallas import tpu as pltpu
from jax.experimental.pallas import tpu_sc as plsc

V = 32
C = 128
L = 16
KC = 8
M_SC = 4
BS = 4


def _make_sc_body(rounds):
    def _sc_body(counts_hbm, pc_hbm, out_hbm, rs_hbm, buf_a, buf_b, pc_v,
                 rs_buf, in_a, in_b, out_a, out_b):
        c = lax.axis_index("c")
        s = lax.axis_index("s")
        k0 = s * KC

        pltpu.sync_copy(pc_hbm, pc_v)
        pc_s = pc_v[...][0]
        lane_ids = lax.iota(jnp.int32, L)

        def lane_total(vec):
            vals = [vec[i] for i in range(L)]
            while len(vals) > 1:
                vals = [a + b for a, b in zip(vals[::2], vals[1::2])]
            return vals[0]

        def in_slice(r):
            i = r * 2 + c
            return counts_hbm.at[i, :, pl.ds(k0, KC), :]

        def out_slice(r):
            i = r * 2 + c
            return out_hbm.at[i, :, pl.ds(k0, KC), :]

        def compute(buf, r):
            rs_vec = jnp.zeros((L,), jnp.float32)
            for k in range(KC):
                def sum_j(j, acc):
                    t = acc
                    for lv in range(C // L):
                        t = t + buf[j, k, pl.ds(lv * L, L)]
                    return t

                acc = lax.fori_loop(0, V, sum_j, jnp.zeros((L,), jnp.float32))
                rs_val = lane_total(acc) + pc_s * jnp.float32(V * C)
                rs_val_vec = jnp.full((L,), rs_val)
                denom = jnp.where(rs_val_vec > 0, rs_val_vec, jnp.float32(1.0))
                recip = jnp.full((L,), 1.0, jnp.float32) / denom
                pr = pc_v[...] * recip
                rs_vec = jnp.where(lane_ids == k, rs_val_vec, rs_vec)

                def scale_j(j, carry):
                    for lv in range(C // L):
                        sl = pl.ds(lv * L, L)
                        buf[j, k, sl] = buf[j, k, sl] * recip + pr
                    return carry

                lax.fori_loop(0, V, scale_j, 0)
            rs_buf[r] = rs_vec

        pltpu.async_copy(in_slice(0), buf_a, in_a)
        pltpu.async_copy(in_slice(1), buf_b, in_b)

        def round_pair(rr, carry):
            r0 = rr * 2
            r1 = r0 + 1
            pltpu.make_async_copy(in_slice(r0), buf_a, in_a).wait()
            compute(buf_a, r0)
            oa = pltpu.async_copy(buf_a, out_slice(r0), out_a)
            pltpu.make_async_copy(in_slice(r1), buf_b, in_b).wait()
            compute(buf_b, r1)
            ob = pltpu.async_copy(buf_b, out_slice(r1), out_b)

            @pl.when(rr < rounds // 2 - 1)
            def _prefetch():
                oa.wait()
                pltpu.async_copy(in_slice(r0 + 2), buf_a, in_a)
                ob.wait()
                pltpu.async_copy(in_slice(r1 + 2), buf_b, in_b)

            return carry

        lax.fori_loop(0, rounds // 2, round_pair, 0)
        pltpu.make_async_copy(buf_a, out_slice(rounds - 2), out_a).wait()
        pltpu.make_async_copy(buf_b, out_slice(rounds - 1), out_b).wait()
        pltpu.sync_copy(rs_buf, rs_hbm.at[:, c, s, :])

    return _sc_body


def _sc_normalize(counts_sc, pcv):
    rounds = M_SC // 2
    mesh = plsc.VectorSubcoreMesh(core_axis_name="c", subcore_axis_name="s")
    k = functools.partial(
        pl.kernel,
        mesh=mesh,
        out_type=[
            jax.ShapeDtypeStruct((M_SC, V, C, C), jnp.float32),
            jax.ShapeDtypeStruct((rounds, 2, V // 2, L), jnp.float32),
        ],
        scratch_types=[
            pltpu.VMEM((V, KC, C), jnp.float32),
            pltpu.VMEM((V, KC, C), jnp.float32),
            pltpu.VMEM((L,), jnp.float32),
            pltpu.VMEM((rounds, L), jnp.float32),
            pltpu.SemaphoreType.DMA,
            pltpu.SemaphoreType.DMA,
            pltpu.SemaphoreType.DMA,
            pltpu.SemaphoreType.DMA,
        ],
    )(_make_sc_body(rounds))
    out, rs = k(counts_sc, pcv)
    return out, rs[..., :KC].reshape(-1)


def _tc_block(pc_ref, counts_ref, out_ref, rs_ref):
    x = counts_ref[...]
    pc = pc_ref[0, 0]
    rs = x.sum(axis=1).sum(axis=2) + pc * jnp.float32(V * C)
    denom = jnp.where(rs > 0, rs, jnp.float32(1.0))
    recip = jnp.float32(1.0) / denom
    out_ref[...] = x * recip[:, None, :, None] + (pc * recip)[:, None, :, None]
    rs_ref[:, 0, :] = rs


def _tc_normalize(counts, pc2d):
    n = V - M_SC
    off = M_SC // BS
    out, rs = pl.pallas_call(
        _tc_block,
        grid=(n // BS,),
        in_specs=[
            pl.BlockSpec(memory_space=pltpu.SMEM),
            pl.BlockSpec((BS, V, C, C), lambda i: (i + off, 0, 0, 0)),
        ],
        out_specs=[
            pl.BlockSpec((BS, V, C, C), lambda i: (i + off, 0, 0, 0)),
            pl.BlockSpec((BS, 1, C), lambda i: (i, 0, 0)),
        ],
        out_shape=[
            jax.ShapeDtypeStruct((V, V, C, C), jnp.float32),
            jax.ShapeDtypeStruct((n, 1, C), jnp.float32),
        ],
    )(pc2d, counts)
    return out, rs.reshape(-1)


def _copy_block(sc_ref, full_ref, out_ref):
    del full_ref
    out_ref[...] = sc_ref[...]


def _merge(sc_out, tc_out):
    return pl.pallas_call(
        _copy_block,
        grid=(1,),
        in_specs=[
            pl.BlockSpec((M_SC, V, C, C), lambda i: (0, 0, 0, 0)),
            pl.BlockSpec(memory_space=pl.ANY),
        ],
        out_specs=pl.BlockSpec((M_SC, V, C, C), lambda i: (0, 0, 0, 0)),
        out_shape=jax.ShapeDtypeStruct((V, V, C, C), jnp.float32),
        input_output_aliases={1: 0},
    )(sc_out, tc_out)


def kernel(transition_counts, pseudocount, hidden_states):
    del hidden_states
    counts = transition_counts.reshape(V, V, C, C)
    pcv = jnp.full((L,), pseudocount, jnp.float32)
    pc2d = jnp.asarray(pseudocount, jnp.float32).reshape(1, 1)
    sc_out, sc_rs = _sc_normalize(counts[:M_SC], pcv)
    tc_out, tc_rs = _tc_normalize(counts, pc2d)
    out = _merge(sc_out, tc_out)
    rs = jnp.concatenate([sc_rs, tc_rs])
    return out.reshape(-1), rs


# speedup vs baseline: 1.9998x; 1.0276x over previous
import functools

import jax
import jax.numpy as jnp
from jax import lax
from jax.experimental import pallas as pl
from jax.experimental.pallas import tpu as pltpu
from jax.experimental.pallas import tpu_sc as plsc

V = 32
C = 128
L = 16
KC = 8
M_SC = 4
BS = 4


def _make_sc_body(rounds, ncores):
    def _sc_body(counts_hbm, pc_hbm, out_hbm, rs_hbm, buf_a, buf_b, pc_v,
                 rs_buf, in_a, in_b, out_a, out_b):
        c = lax.axis_index("c")
        s = lax.axis_index("s")
        k0 = s * KC

        pltpu.sync_copy(pc_hbm, pc_v)
        pc_s = pc_v[...][0]
        lane_ids = lax.iota(jnp.int32, L)

        def lane_total(vec):
            vals = [vec[i] for i in range(L)]
            while len(vals) > 1:
                vals = [a + b for a, b in zip(vals[::2], vals[1::2])]
            return vals[0]

        def in_slice(r):
            i = r * ncores + c
            return counts_hbm.at[i, :, pl.ds(k0, KC), :]

        def out_slice(r):
            i = r * ncores + c
            return out_hbm.at[i, :, pl.ds(k0, KC), :]

        def compute(buf, r):
            rs_vec = jnp.zeros((L,), jnp.float32)
            for k in range(KC):
                def sum_j(j, acc):
                    t = acc
                    for lv in range(C // L):
                        t = t + buf[j, k, pl.ds(lv * L, L)]
                    return t

                acc = lax.fori_loop(0, V, sum_j, jnp.zeros((L,), jnp.float32))
                rs_val = lane_total(acc) + pc_s * jnp.float32(V * C)
                rs_val_vec = jnp.full((L,), rs_val)
                denom = jnp.where(rs_val_vec > 0, rs_val_vec, jnp.float32(1.0))
                recip = jnp.full((L,), 1.0, jnp.float32) / denom
                pr = pc_v[...] * recip
                rs_vec = jnp.where(lane_ids == k, rs_val_vec, rs_vec)

                def scale_j(j, carry):
                    for lv in range(C // L):
                        sl = pl.ds(lv * L, L)
                        buf[j, k, sl] = buf[j, k, sl] * recip + pr
                    return carry

                lax.fori_loop(0, V, scale_j, 0)
            rs_buf[r] = rs_vec

        pltpu.async_copy(in_slice(0), buf_a, in_a)
        pltpu.async_copy(in_slice(1), buf_b, in_b)

        def round_pair(rr, carry):
            r0 = rr * 2
            r1 = r0 + 1
            pltpu.make_async_copy(in_slice(r0), buf_a, in_a).wait()
            compute(buf_a, r0)
            oa = pltpu.async_copy(buf_a, out_slice(r0), out_a)
            pltpu.make_async_copy(in_slice(r1), buf_b, in_b).wait()
            compute(buf_b, r1)
            ob = pltpu.async_copy(buf_b, out_slice(r1), out_b)

            @pl.when(rr < rounds // 2 - 1)
            def _prefetch():
                oa.wait()
                pltpu.async_copy(in_slice(r0 + 2), buf_a, in_a)
                ob.wait()
                pltpu.async_copy(in_slice(r1 + 2), buf_b, in_b)

            return carry

        lax.fori_loop(0, rounds // 2, round_pair, 0)
        pltpu.make_async_copy(buf_a, out_slice(rounds - 2), out_a).wait()
        pltpu.make_async_copy(buf_b, out_slice(rounds - 1), out_b).wait()
        pltpu.sync_copy(rs_buf, rs_hbm.at[:, c, s, :])

    return _sc_body


N_SC_CORES = 1


def _sc_normalize(counts_sc, pcv):
    rounds = M_SC // N_SC_CORES
    mesh = plsc.VectorSubcoreMesh(
        core_axis_name="c", subcore_axis_name="s", num_cores=N_SC_CORES
    )
    k = functools.partial(
        pl.kernel,
        mesh=mesh,
        out_type=[
            jax.ShapeDtypeStruct((M_SC, V, C, C), jnp.float32),
            jax.ShapeDtypeStruct((rounds, N_SC_CORES, V // 2, L), jnp.float32),
        ],
        scratch_types=[
            pltpu.VMEM((V, KC, C), jnp.float32),
            pltpu.VMEM((V, KC, C), jnp.float32),
            pltpu.VMEM((L,), jnp.float32),
            pltpu.VMEM((rounds, L), jnp.float32),
            pltpu.SemaphoreType.DMA,
            pltpu.SemaphoreType.DMA,
            pltpu.SemaphoreType.DMA,
            pltpu.SemaphoreType.DMA,
        ],
    )(_make_sc_body(rounds, N_SC_CORES))
    out, rs = k(counts_sc, pcv)
    return out, rs[..., :KC].reshape(-1)


def _tc_block(pc_ref, counts_ref, out_ref, rs_ref):
    x = counts_ref[...]
    pc = pc_ref[0, 0]
    rs = x.sum(axis=1).sum(axis=2) + pc * jnp.float32(V * C)
    denom = jnp.where(rs > 0, rs, jnp.float32(1.0))
    recip = jnp.float32(1.0) / denom
    out_ref[...] = x * recip[:, None, :, None] + (pc * recip)[:, None, :, None]
    rs_ref[:, 0, :] = rs


def _tc_normalize(counts, pc2d):
    n = V - M_SC
    off = M_SC // BS
    out, rs = pl.pallas_call(
        _tc_block,
        grid=(n // BS,),
        in_specs=[
            pl.BlockSpec(memory_space=pltpu.SMEM),
            pl.BlockSpec((BS, V, C, C), lambda i: (i + off, 0, 0, 0)),
        ],
        out_specs=[
            pl.BlockSpec((BS, V, C, C), lambda i: (i + off, 0, 0, 0)),
            pl.BlockSpec((BS, 1, C), lambda i: (i, 0, 0)),
        ],
        out_shape=[
            jax.ShapeDtypeStruct((V, V, C, C), jnp.float32),
            jax.ShapeDtypeStruct((n, 1, C), jnp.float32),
        ],
    )(pc2d, counts)
    return out, rs.reshape(-1)


def _copy_block(sc_ref, full_ref, out_ref):
    del full_ref
    out_ref[...] = sc_ref[...]


def _merge(sc_out, tc_out):
    return pl.pallas_call(
        _copy_block,
        grid=(1,),
        in_specs=[
            pl.BlockSpec((M_SC, V, C, C), lambda i: (0, 0, 0, 0)),
            pl.BlockSpec(memory_space=pl.ANY),
        ],
        out_specs=pl.BlockSpec((M_SC, V, C, C), lambda i: (0, 0, 0, 0)),
        out_shape=jax.ShapeDtypeStruct((V, V, C, C), jnp.float32),
        input_output_aliases={1: 0},
    )(sc_out, tc_out)


def kernel(transition_counts, pseudocount, hidden_states):
    del hidden_states
    counts = transition_counts.reshape(V, V, C, C)
    pcv = jnp.full((L,), pseudocount, jnp.float32)
    pc2d = jnp.asarray(pseudocount, jnp.float32).reshape(1, 1)
    sc_out, sc_rs = _sc_normalize(counts[:M_SC], pcv)
    tc_out, tc_rs = _tc_normalize(counts, pc2d)
    out = _merge(sc_out, tc_out)
    rs = jnp.concatenate([sc_rs, tc_rs])
    return out.reshape(-1), rs
